# Initial kernel scaffold; baseline (speedup 1.0000x reference)
import jax
import jax.numpy as jnp
from jax.experimental import pallas as pl


def kernel(x, edge_index, params):
    raise NotImplementedError("write your pallas kernel here")


# speedup vs baseline: 21.6503x; 21.6503x over previous
import functools

import jax
import jax.numpy as jnp
from jax import lax
from jax.experimental import pallas as pl
from jax.experimental.pallas import tpu as pltpu
from jax.experimental.pallas import tpu_sc as plsc

N_NODES = 10000
N_PAD = 10240
E_EDGES = 320000
D = 128
HD = D // 2
NC, NS, LANES = 2, 16, 16
EPT = E_EDGES // NS
CHUNK = 80
NCHUNK = EPT // CHUNK
ROWS_PT = N_PAD // NS
M_BLK = 1024
GRID = N_PAD // M_BLK

_NEG = -3.0e38


def _proj(h, asv, adv, as2_ref, ad2_ref, mx_ref, i):
    asr = jnp.sum(h * asv, axis=1)
    adr = jnp.sum(h * adv, axis=1)
    as2_ref[...] = asr.reshape(8, 128)
    ad2_ref[...] = adr.reshape(8, 128)
    cur = jnp.max(asr)
    prev = jnp.where(i == 0, _NEG, mx_ref[0, 0])
    mx_ref[...] = jnp.full((8, 128), jnp.maximum(prev, cur), dtype=jnp.float32)


def _dense_first_body(x_ref, w_ref, asv_ref, adv_ref,
                      h_ref, as2_ref, ad2_ref, mx_ref):
    h = jnp.dot(x_ref[...], w_ref[...], preferred_element_type=jnp.float32)
    h_ref[...] = h
    _proj(h, asv_ref[...], adv_ref[...], as2_ref, ad2_ref, mx_ref,
          pl.program_id(0))


def _combine(p0, p1, s_part, b):
    rows = jnp.concatenate([p0, p1], axis=1)
    s = jnp.sum(s_part, axis=0).reshape(M_BLK, 1)
    return jnp.maximum(rows / (s + 1e-16) + b, 0.0)


def _dense_mid_body(p0_ref, p1_ref, s_ref, b_ref, w_ref, asv_ref, adv_ref,
                    h_ref, as2_ref, ad2_ref, mx_ref):
    x = _combine(p0_ref[...], p1_ref[...], s_ref[...], b_ref[...])
    h = jnp.dot(x, w_ref[...], preferred_element_type=jnp.float32)
    h_ref[...] = h
    _proj(h, asv_ref[...], adv_ref[...], as2_ref, ad2_ref, mx_ref,
          pl.program_id(0))


def _combine_last_body(p0_ref, p1_ref, s_ref, b_ref, y_ref):
    y_ref[...] = _combine(p0_ref[...], p1_ref[...], s_ref[...], b_ref[...])


_row_spec = pl.BlockSpec((M_BLK, D), lambda i: (i, 0))
_half_spec = pl.BlockSpec((M_BLK, HD), lambda i: (i, 0))
_s_spec = pl.BlockSpec((NS, M_BLK), lambda i: (0, i))
_vec_spec = pl.BlockSpec((1, D), lambda i: (0, 0))
_w_spec = pl.BlockSpec((D, D), lambda i: (0, 0))
_a2_spec = pl.BlockSpec((8, 128), lambda i: (i, 0))
_mx_spec = pl.BlockSpec((8, 128), lambda i: (0, 0))

_dense_outs = (
    jax.ShapeDtypeStruct((N_PAD, D), jnp.float32),
    jax.ShapeDtypeStruct((80, 128), jnp.float32),
    jax.ShapeDtypeStruct((80, 128), jnp.float32),
    jax.ShapeDtypeStruct((8, 128), jnp.float32),
)

_dense_first = pl.pallas_call(
    _dense_first_body,
    grid=(GRID,),
    in_specs=[_row_spec, _w_spec, _vec_spec, _vec_spec],
    out_specs=[_row_spec, _a2_spec, _a2_spec, _mx_spec],
    out_shape=_dense_outs,
)

_dense_mid = pl.pallas_call(
    _dense_mid_body,
    grid=(GRID,),
    in_specs=[_half_spec, _half_spec, _s_spec, _vec_spec, _w_spec,
              _vec_spec, _vec_spec],
    out_specs=[_row_spec, _a2_spec, _a2_spec, _mx_spec],
    out_shape=_dense_outs,
)

_combine_last = pl.pallas_call(
    _combine_last_body,
    grid=(GRID,),
    in_specs=[_half_spec, _half_spec, _s_spec, _vec_spec],
    out_specs=_row_spec,
    out_shape=jax.ShapeDtypeStruct((N_PAD, D), jnp.float32),
)


_sc_mesh = plsc.VectorSubcoreMesh(
    core_axis_name="c", subcore_axis_name="s", num_cores=NC, num_subcores=NS)


@functools.partial(
    pl.kernel,
    out_type=(jax.ShapeDtypeStruct((NC, N_PAD, HD), jnp.float32),
              jax.ShapeDtypeStruct((NS, N_PAD), jnp.float32)),
    mesh=_sc_mesh,
    compiler_params=pltpu.CompilerParams(
        needs_layout_passes=False, use_tc_tiling_on_sc=False),
    scratch_types=[
        pltpu.VMEM((N_PAD,), jnp.float32),
        pltpu.VMEM((N_PAD,), jnp.float32),
        pltpu.VMEM((NCHUNK, CHUNK), jnp.int32),
        pltpu.VMEM((NCHUNK, CHUNK), jnp.int32),
        pltpu.VMEM((CHUNK,), jnp.int32),
        pltpu.VMEM((CHUNK,), jnp.float32),
        pltpu.VMEM((CHUNK, HD), jnp.float32),
        pltpu.VMEM((N_PAD,), jnp.float32),
        pltpu.VMEM((LANES,), jnp.float32),
        pltpu.VMEM_SHARED((N_PAD, HD), jnp.float32),
        pltpu.SemaphoreType.DMA,
    ],
)
def _sc_attn(h2_hbm, src_hbm, dst_hbm, asrc_hbm, adst_hbm, mx_hbm,
             out_hbm, s_hbm,
             asrc_v, adst_v, src_v, dst_v, gidx_v, w_v, rows_v, s_v, mx_v,
             acc_sh, sem):
    cid = lax.axis_index("c")
    sid = lax.axis_index("s")

    pltpu.sync_copy(asrc_hbm, asrc_v)
    pltpu.sync_copy(adst_hbm, adst_v)
    pltpu.sync_copy(src_hbm.at[sid], src_v)
    pltpu.sync_copy(dst_hbm.at[sid], dst_v)
    pltpu.sync_copy(mx_hbm, mx_v)

    zero = jnp.zeros((LANES,), jnp.float32)

    def _zrow(r, carry):
        for q in range(HD // LANES):
            rows_v[r, pl.ds(q * LANES, LANES)] = zero
        return carry

    lax.fori_loop(0, CHUNK, _zrow, 0)
    for k in range(ROWS_PT // CHUNK):
        pltpu.sync_copy(
            rows_v, acc_sh.at[pl.ds(sid * ROWS_PT + k * CHUNK, CHUNK)])

    def _zs(r, carry):
        s_v[pl.ds(r * LANES, LANES)] = zero
        return carry

    lax.fori_loop(0, N_PAD // LANES, _zs, 0)
    plsc.subcore_barrier()

    mxv = mx_v[...]

    def _chunk(j, carry):
        for g in range(CHUNK // LANES):
            si = src_v[j, pl.ds(g * LANES, LANES)]
            di = dst_v[j, pl.ds(g * LANES, LANES)]
            a_s = plsc.load_gather(asrc_v, [si])
            a_d = plsc.load_gather(adst_v, [di])
            z = a_s + a_d
            logit = jnp.where(z >= 0.0, z, 0.2 * z)
            zs = mxv + a_d
            shift = jnp.where(zs >= 0.0, zs, 0.2 * zs)
            w = jnp.exp(logit - shift)
            w_v[pl.ds(g * LANES, LANES)] = w
            plsc.addupdate_scatter(s_v, [di], w)
            gidx_v[pl.ds(g * LANES, LANES)] = si * 2 + cid

        pltpu.async_copy(h2_hbm.at[gidx_v], rows_v, sem).wait()

        def _scale(e, c2):
            wsp = plsc.load_gather(
                w_v, [jnp.zeros((LANES,), jnp.int32) + e])
            for q in range(HD // LANES):
                rows_v[e, pl.ds(q * LANES, LANES)] = (
                    rows_v[e, pl.ds(q * LANES, LANES)] * wsp)
            return c2

        lax.fori_loop(0, CHUNK, _scale, 0)
        pltpu.sync_copy(rows_v, acc_sh.at[dst_v.at[j]], add=True)
        return carry

    lax.fori_loop(0, NCHUNK, _chunk, 0)

    @pl.when(cid == 0)
    def _():
        pltpu.sync_copy(s_v, s_hbm.at[sid])

    plsc.subcore_barrier()

    pltpu.sync_copy(acc_sh.at[pl.ds(sid * ROWS_PT, ROWS_PT)],
                    out_hbm.at[cid, pl.ds(sid * ROWS_PT, ROWS_PT)])


def kernel(x, edge_index, params):
    x = x.reshape(-1, D).astype(jnp.float32)
    edge_index = edge_index.reshape(2, -1)
    xp = jnp.pad(x, ((0, N_PAD - N_NODES), (0, 0)))
    src = edge_index[0].reshape(NS, NCHUNK, CHUNK)
    dst = edge_index[1].reshape(NS, NCHUNK, CHUNK)

    acc = spart = None
    for i, (W, a_src, a_dst, b) in enumerate(params):
        asv = a_src.reshape(1, D)
        adv = a_dst.reshape(1, D)
        if i == 0:
            h, as2, ad2, mx8 = _dense_first(xp, W, asv, adv)
        else:
            h, as2, ad2, mx8 = _dense_mid(
                acc[0], acc[1], spart, prev_b.reshape(1, D), W, asv, adv)
        asrc = as2.reshape(N_PAD)
        adst = ad2.reshape(N_PAD)
        mx16 = mx8[0, :LANES]
        h2 = h.reshape(2 * N_PAD, HD)
        acc, spart = _sc_attn(h2, src, dst, asrc, adst, mx16)
        prev_b = b

    y = _combine_last(acc[0], acc[1], spart, prev_b.reshape(1, D))
    return y[:N_NODES]


# speedup vs baseline: 37.2101x; 1.7187x over previous
import functools

import jax
import jax.numpy as jnp
from jax import lax
from jax.experimental import pallas as pl
from jax.experimental.pallas import tpu as pltpu
from jax.experimental.pallas import tpu_sc as plsc

N_NODES = 10000
N_PAD = 10240
E_EDGES = 320000
D = 128
HD = D // 2
NC, NS, LANES = 2, 16, 16
EPT = E_EDGES // NS
CHUNK = 80
NCHUNK = EPT // CHUNK
ROWS_PT = N_PAD // NS
M_BLK = 1024
GRID = N_PAD // M_BLK

_NEG = -3.0e38


def _proj(h, asv, adv, as2_ref, ad2_ref, mx_ref, i):
    asr = jnp.sum(h * asv, axis=1)
    adr = jnp.sum(h * adv, axis=1)
    as2_ref[...] = asr.reshape(8, 128)
    ad2_ref[...] = adr.reshape(8, 128)
    cur = jnp.max(asr)
    prev = jnp.where(i == 0, _NEG, mx_ref[0, 0])
    mx_ref[...] = jnp.full((8, 128), jnp.maximum(prev, cur), dtype=jnp.float32)


def _dense_first_body(x_ref, w_ref, asv_ref, adv_ref,
                      h_ref, as2_ref, ad2_ref, mx_ref):
    h = jnp.dot(x_ref[...], w_ref[...], preferred_element_type=jnp.float32)
    h_ref[...] = h
    _proj(h, asv_ref[...], adv_ref[...], as2_ref, ad2_ref, mx_ref,
          pl.program_id(0))


def _combine(p0, p1, s_part, b):
    rows = jnp.concatenate([p0, p1], axis=1)
    s = jnp.sum(s_part, axis=0).reshape(M_BLK, 1)
    return jnp.maximum(rows / (s + 1e-16) + b, 0.0)


def _dense_mid_body(p0_ref, p1_ref, s_ref, b_ref, w_ref, asv_ref, adv_ref,
                    h_ref, as2_ref, ad2_ref, mx_ref):
    x = _combine(p0_ref[...], p1_ref[...], s_ref[...], b_ref[...])
    h = jnp.dot(x, w_ref[...], preferred_element_type=jnp.float32)
    h_ref[...] = h
    _proj(h, asv_ref[...], adv_ref[...], as2_ref, ad2_ref, mx_ref,
          pl.program_id(0))


def _combine_last_body(p0_ref, p1_ref, s_ref, b_ref, y_ref):
    y_ref[...] = _combine(p0_ref[...], p1_ref[...], s_ref[...], b_ref[...])


_row_spec = pl.BlockSpec((M_BLK, D), lambda i: (i, 0))
_half_spec = pl.BlockSpec((M_BLK, HD), lambda i: (i, 0))
_s_spec = pl.BlockSpec((NS, M_BLK), lambda i: (0, i))
_vec_spec = pl.BlockSpec((1, D), lambda i: (0, 0))
_w_spec = pl.BlockSpec((D, D), lambda i: (0, 0))
_a2_spec = pl.BlockSpec((8, 128), lambda i: (i, 0))
_mx_spec = pl.BlockSpec((8, 128), lambda i: (0, 0))

_dense_outs = (
    jax.ShapeDtypeStruct((N_PAD, D), jnp.float32),
    jax.ShapeDtypeStruct((80, 128), jnp.float32),
    jax.ShapeDtypeStruct((80, 128), jnp.float32),
    jax.ShapeDtypeStruct((8, 128), jnp.float32),
)

_dense_first = pl.pallas_call(
    _dense_first_body,
    grid=(GRID,),
    in_specs=[_row_spec, _w_spec, _vec_spec, _vec_spec],
    out_specs=[_row_spec, _a2_spec, _a2_spec, _mx_spec],
    out_shape=_dense_outs,
)

_dense_mid = pl.pallas_call(
    _dense_mid_body,
    grid=(GRID,),
    in_specs=[_half_spec, _half_spec, _s_spec, _vec_spec, _w_spec,
              _vec_spec, _vec_spec],
    out_specs=[_row_spec, _a2_spec, _a2_spec, _mx_spec],
    out_shape=_dense_outs,
)

_combine_last = pl.pallas_call(
    _combine_last_body,
    grid=(GRID,),
    in_specs=[_half_spec, _half_spec, _s_spec, _vec_spec],
    out_specs=_row_spec,
    out_shape=jax.ShapeDtypeStruct((N_PAD, D), jnp.float32),
)


_sc_mesh = plsc.VectorSubcoreMesh(
    core_axis_name="c", subcore_axis_name="s", num_cores=NC, num_subcores=NS)


@functools.partial(
    pl.kernel,
    out_type=(jax.ShapeDtypeStruct((NC, N_PAD, HD), jnp.float32),
              jax.ShapeDtypeStruct((NS, N_PAD), jnp.float32)),
    mesh=_sc_mesh,
    compiler_params=pltpu.CompilerParams(
        needs_layout_passes=False, use_tc_tiling_on_sc=False),
    scratch_types=[
        pltpu.VMEM((N_PAD,), jnp.float32),
        pltpu.VMEM((N_PAD,), jnp.float32),
        pltpu.VMEM((NCHUNK, CHUNK), jnp.int32),
        pltpu.VMEM((NCHUNK, CHUNK), jnp.int32),
        pltpu.VMEM((2, CHUNK), jnp.int32),
        pltpu.VMEM((2, CHUNK), jnp.float32),
        pltpu.VMEM((2, CHUNK, HD), jnp.float32),
        pltpu.VMEM((N_PAD,), jnp.float32),
        pltpu.VMEM((LANES,), jnp.float32),
        pltpu.VMEM_SHARED((N_PAD, HD), jnp.float32),
        pltpu.SemaphoreType.DMA,
        pltpu.SemaphoreType.DMA,
    ],
)
def _sc_attn(h2_hbm, src_hbm, dst_hbm, asrc_hbm, adst_hbm, mx_hbm,
             out_hbm, s_hbm,
             asrc_v, adst_v, src_v, dst_v, gidx_v, w_v, rows_v, s_v, mx_v,
             acc_sh, sem_g, sem_s):
    cid = lax.axis_index("c")
    sid = lax.axis_index("s")

    pltpu.sync_copy(asrc_hbm, asrc_v)
    pltpu.sync_copy(adst_hbm, adst_v)
    pltpu.sync_copy(src_hbm.at[sid], src_v)
    pltpu.sync_copy(dst_hbm.at[sid], dst_v)
    pltpu.sync_copy(mx_hbm, mx_v)

    zero = jnp.zeros((LANES,), jnp.float32)

    def _zrow(r, carry):
        for q in range(HD // LANES):
            rows_v[0, r, pl.ds(q * LANES, LANES)] = zero
        return carry

    lax.fori_loop(0, CHUNK, _zrow, 0)
    for k in range(ROWS_PT // CHUNK):
        pltpu.sync_copy(
            rows_v.at[0],
            acc_sh.at[pl.ds(sid * ROWS_PT + k * CHUNK, CHUNK)])

    def _zs(r, carry):
        s_v[pl.ds(r * LANES, LANES)] = zero
        return carry

    lax.fori_loop(0, N_PAD // LANES, _zs, 0)
    plsc.subcore_barrier()

    mxv = mx_v[...]

    def _wphase(j, p):
        for g in range(CHUNK // LANES):
            si = src_v[j, pl.ds(g * LANES, LANES)]
            di = dst_v[j, pl.ds(g * LANES, LANES)]
            a_s = plsc.load_gather(asrc_v, [si])
            a_d = plsc.load_gather(adst_v, [di])
            z = a_s + a_d
            logit = jnp.where(z >= 0.0, z, 0.2 * z)
            zs = mxv + a_d
            shift = jnp.where(zs >= 0.0, zs, 0.2 * zs)
            w = jnp.exp(logit - shift)
            w_v[p, pl.ds(g * LANES, LANES)] = w
            plsc.addupdate_scatter(s_v, [di], w)
            gidx_v[p, pl.ds(g * LANES, LANES)] = si * 2 + cid

    _wphase(0, 0)
    pltpu.async_copy(h2_hbm.at[gidx_v.at[0]], rows_v.at[0], sem_g)

    def _chunk(j, carry):
        p = j % 2
        q = 1 - p
        pltpu.make_async_copy(
            h2_hbm.at[gidx_v.at[p]], rows_v.at[p], sem_g).wait()

        @pl.when(j > 0)
        def _():
            pltpu.make_async_copy(
                rows_v.at[q], acc_sh.at[dst_v.at[j - 1]], sem_s).wait()

        @pl.when(j + 1 < NCHUNK)
        def _():
            _wphase(j + 1, q)
            pltpu.async_copy(h2_hbm.at[gidx_v.at[q]], rows_v.at[q], sem_g)

        @plsc.parallel_loop(0, CHUNK, unroll=4)
        def _scale(e):
            wsp = plsc.load_gather(
                w_v.at[p], [jnp.zeros((LANES,), jnp.int32) + e])
            for q2 in range(HD // LANES):
                rows_v[p, e, pl.ds(q2 * LANES, LANES)] = (
                    rows_v[p, e, pl.ds(q2 * LANES, LANES)] * wsp)

        pltpu.async_copy(
            rows_v.at[p], acc_sh.at[dst_v.at[j]], sem_s, add=True)
        return carry

    lax.fori_loop(0, NCHUNK, _chunk, 0)
    pltpu.make_async_copy(
        rows_v.at[(NCHUNK - 1) % 2],
        acc_sh.at[dst_v.at[NCHUNK - 1]], sem_s).wait()

    @pl.when(cid == 0)
    def _():
        pltpu.sync_copy(s_v, s_hbm.at[sid])

    plsc.subcore_barrier()

    pltpu.sync_copy(acc_sh.at[pl.ds(sid * ROWS_PT, ROWS_PT)],
                    out_hbm.at[cid, pl.ds(sid * ROWS_PT, ROWS_PT)])


def kernel(x, edge_index, params):
    x = x.reshape(-1, D).astype(jnp.float32)
    edge_index = edge_index.reshape(2, -1)
    xp = jnp.pad(x, ((0, N_PAD - N_NODES), (0, 0)))
    src = edge_index[0].reshape(NS, NCHUNK, CHUNK)
    dst = edge_index[1].reshape(NS, NCHUNK, CHUNK)

    acc = spart = None
    for i, (W, a_src, a_dst, b) in enumerate(params):
        asv = a_src.reshape(1, D)
        adv = a_dst.reshape(1, D)
        if i == 0:
            h, as2, ad2, mx8 = _dense_first(xp, W, asv, adv)
        else:
            h, as2, ad2, mx8 = _dense_mid(
                acc[0], acc[1], spart, prev_b.reshape(1, D), W, asv, adv)
        asrc = as2.reshape(N_PAD)
        adst = ad2.reshape(N_PAD)
        mx16 = mx8[0, :LANES]
        h2 = h.reshape(2 * N_PAD, HD)
        acc, spart = _sc_attn(h2, src, dst, asrc, adst, mx16)
        prev_b = b

    y = _combine_last(acc[0], acc[1], spart, prev_b.reshape(1, D))
    return y[:N_NODES]
